# baseline (speedup 1.0000x reference)
import functools

import jax
import jax.numpy as jnp
from jax import lax
from jax.experimental import pallas as pl
from jax.experimental.pallas import tpu as pltpu
from jax.experimental.pallas import tpu_sc as plsc

_N = 50000
_C = 32
_K = 27
_H = _C // 2
_G0 = 14

_NC = 2
_NS = 16
_NW = _NC * _NS
_SUB = 208
_NSUB = 8
_CH = _SUB * _NSUB

_BLKN = 2000
_NBLK = _N // _BLKN
_LNB = 2000
_NLNB = _N // _LNB


def _mat_body(kg, feats_ref, wconv_ref, wlin_ref, out_ref, wlo_ref, whi_ref):
    @pl.when(pl.program_id(0) == 0)
    def _():
        w = lax.dot_general(wconv_ref[...], wlin_ref[...],
                            (((2,), (1,)), ((), ())),
                            preferred_element_type=jnp.float32)
        for k in range(kg):
            wlo_ref[pl.ds(0, _C), pl.ds(k * _H, _H)] = w[k][:, :_H]
            whi_ref[pl.ds(0, _C), pl.ds(k * _H, _H)] = w[k][:, _H:]

    y_lo = jnp.dot(feats_ref[...], wlo_ref[...],
                   preferred_element_type=jnp.float32)
    y_hi = jnp.dot(feats_ref[...], whi_ref[...],
                   preferred_element_type=jnp.float32)
    bl = lax.bitcast_convert_type(y_lo, jnp.uint32)
    bh = lax.bitcast_convert_type(y_hi, jnp.uint32)
    rl = (bl + jnp.uint32(0x7FFF) + ((bl >> 16) & jnp.uint32(1))) >> 16
    rh = (bh + jnp.uint32(0x7FFF) + ((bh >> 16) & jnp.uint32(1))) \
        & jnp.uint32(0xFFFF0000)
    out_ref[...] = lax.bitcast_convert_type(rl | rh, jnp.int32)


def _sc_body(kg, g0, tbl_hbm, idx_hbm, out_hbm, idx_v, stg_v, acc_v, sem):
    c = lax.axis_index("c")
    s = lax.axis_index("s")
    wid = s * _NC + c
    base = jnp.minimum(wid * _CH, _N - _CH)

    shift = jnp.full((16,), 16, dtype=jnp.int32)
    mask = jnp.full((16,), -65536, dtype=jnp.int32)
    kmul = jnp.full((16,), kg, dtype=jnp.int32)

    for sub in range(_NSUB):
        pltpu.sync_copy(
            idx_hbm.at[pl.ds(g0, kg), pl.ds(base + sub * _SUB, _SUB)], idx_v)

        def conv(j, carry):
            for k in range(kg):
                v = idx_v[k, pl.ds(j * 16, 16)]
                idx_v[k, pl.ds(j * 16, 16)] = v * kmul + k
            return carry

        lax.fori_loop(0, _SUB // 16, conv, 0)
        cps = [
            pltpu.async_copy(tbl_hbm.at[idx_v.at[k]], stg_v.at[k], sem)
            for k in range(kg)
        ]
        for cp in cps:
            cp.wait()

        def row(r, carry):
            acc_lo = jnp.zeros((16,), jnp.float32)
            acc_hi = jnp.zeros((16,), jnp.float32)
            for k in range(kg):
                w = stg_v[k, r, :]
                acc_lo = acc_lo + plsc.bitcast(
                    lax.shift_left(w, shift), jnp.float32)
                acc_hi = acc_hi + plsc.bitcast(
                    lax.bitwise_and(w, mask), jnp.float32)
            acc_v[r, pl.ds(0, 16)] = acc_lo
            acc_v[r, pl.ds(16, 16)] = acc_hi
            return carry

        lax.fori_loop(0, _SUB, row, 0)
        pltpu.sync_copy(acc_v, out_hbm.at[pl.ds(base + sub * _SUB, _SUB)])


def _ln_body(h0_ref, h1_ref, h2_ref, wlin_ref, bconv_ref, blin_ref,
             g_ref, b_ref, out_ref):
    bias = lax.dot_general(bconv_ref[...], wlin_ref[...],
                           (((1,), (1,)), ((), ())),
                           preferred_element_type=jnp.float32) + blin_ref[...]
    x = (h0_ref[...] + h1_ref[...]) + (h2_ref[...] + bias)
    mu = jnp.mean(x, axis=-1, keepdims=True)
    xc = x - mu
    var = jnp.mean(xc * xc, axis=-1, keepdims=True)
    out_ref[...] = xc * lax.rsqrt(var + 1e-5) * g_ref[...] + b_ref[...]


def _make_table(feats, wconv_g, wlin, kg):
    return pl.pallas_call(
        functools.partial(_mat_body, kg),
        grid=(_NBLK,),
        in_specs=[
            pl.BlockSpec((_BLKN, _C), lambda i: (i, 0)),
            pl.BlockSpec((kg, _C, _C), lambda i: (0, 0, 0)),
            pl.BlockSpec((_C, _C), lambda i: (0, 0)),
        ],
        out_specs=pl.BlockSpec((_BLKN, kg * _H), lambda i: (i, 0)),
        out_shape=jax.ShapeDtypeStruct((_N, kg * _H), jnp.int32),
        scratch_shapes=[pltpu.VMEM((_C, kg * _H), jnp.float32),
                        pltpu.VMEM((_C, kg * _H), jnp.float32)],
    )(feats, wconv_g, wlin)


def _gather_group(tbl, idx, kg, g0):
    return pl.kernel(
        functools.partial(_sc_body, kg, g0),
        out_type=jax.ShapeDtypeStruct((_N, _C), jnp.float32),
        mesh=plsc.VectorSubcoreMesh(core_axis_name="c", subcore_axis_name="s"),
        compiler_params=pltpu.CompilerParams(use_tc_tiling_on_sc=False,
                                             needs_layout_passes=False),
        scratch_types=[
            pltpu.VMEM((kg, _SUB), jnp.int32),
            pltpu.VMEM((kg, _SUB, _H), jnp.int32),
            pltpu.VMEM((_SUB, _C), jnp.float32),
            pltpu.SemaphoreType.DMA,
        ],
    )(tbl.reshape(_N * kg, _H), idx)


_GROUPS = (11, 8, 8)


def kernel(feats, neighbor_idx, W_conv, b_conv, W_lin, b_lin, ln_g, ln_b):
    idx = neighbor_idx.astype(jnp.int32)

    hs = []
    g0 = 0
    for kg in _GROUPS:
        tbl = _make_table(feats, W_conv[g0:g0 + kg], W_lin, kg)
        hs.append(_gather_group(tbl, idx, kg, g0))
        g0 += kg

    out = pl.pallas_call(
        _ln_body,
        grid=(_NLNB,),
        in_specs=[
            pl.BlockSpec((_LNB, _C), lambda i: (i, 0)),
            pl.BlockSpec((_LNB, _C), lambda i: (i, 0)),
            pl.BlockSpec((_LNB, _C), lambda i: (i, 0)),
            pl.BlockSpec((_C, _C), lambda i: (0, 0)),
            pl.BlockSpec((1, _C), lambda i: (0, 0)),
            pl.BlockSpec((1, _C), lambda i: (0, 0)),
            pl.BlockSpec((1, _C), lambda i: (0, 0)),
            pl.BlockSpec((1, _C), lambda i: (0, 0)),
        ],
        out_specs=pl.BlockSpec((_LNB, _C), lambda i: (i, 0)),
        out_shape=jax.ShapeDtypeStruct((_N, _C), jnp.float32),
    )(*hs, W_lin, b_conv.reshape(1, _C), b_lin.reshape(1, _C),
      ln_g.reshape(1, _C), ln_b.reshape(1, _C))

    return out


# speedup vs baseline: 1.0692x; 1.0692x over previous
import functools

import jax
import jax.numpy as jnp
from jax import lax
from jax.experimental import pallas as pl
from jax.experimental.pallas import tpu as pltpu
from jax.experimental.pallas import tpu_sc as plsc

_N = 50000
_C = 32
_K = 27
_H = _C // 2
_G0 = 14

_NC = 2
_NS = 16
_NW = _NC * _NS
_SUB = 208
_NSUB = 8
_CH = _SUB * _NSUB

_BLKN = 2000
_NBLK = _N // _BLKN
_LNB = 2000
_NLNB = _N // _LNB


def _mat_body(kg, feats_ref, wconv_ref, wlin_ref, out_ref, wlo_ref, whi_ref):
    @pl.when(pl.program_id(0) == 0)
    def _():
        w = lax.dot_general(wconv_ref[...], wlin_ref[...],
                            (((2,), (1,)), ((), ())),
                            preferred_element_type=jnp.float32)
        for k in range(kg):
            wlo_ref[pl.ds(0, _C), pl.ds(k * _H, _H)] = w[k][:, :_H]
            whi_ref[pl.ds(0, _C), pl.ds(k * _H, _H)] = w[k][:, _H:]

    y_lo = jnp.dot(feats_ref[...], wlo_ref[...],
                   preferred_element_type=jnp.float32)
    y_hi = jnp.dot(feats_ref[...], whi_ref[...],
                   preferred_element_type=jnp.float32)
    bl = lax.bitcast_convert_type(y_lo, jnp.uint32)
    bh = lax.bitcast_convert_type(y_hi, jnp.uint32)
    rl = (bl + jnp.uint32(0x7FFF) + ((bl >> 16) & jnp.uint32(1))) >> 16
    rh = (bh + jnp.uint32(0x7FFF) + ((bh >> 16) & jnp.uint32(1))) \
        & jnp.uint32(0xFFFF0000)
    out_ref[...] = lax.bitcast_convert_type(rl | rh, jnp.int32)


def _sc_body(kg, g0, tbl_hbm, idx_hbm, out_hbm, idx_v, stg_v, acc_v, sem):
    c = lax.axis_index("c")
    s = lax.axis_index("s")
    wid = s * _NC + c
    base = jnp.minimum(wid * _CH, _N - _CH)

    shift = jnp.full((16,), 16, dtype=jnp.int32)
    mask = jnp.full((16,), -65536, dtype=jnp.int32)
    kmul = jnp.full((16,), kg, dtype=jnp.int32)

    for sub in range(_NSUB):
        pltpu.sync_copy(
            idx_hbm.at[pl.ds(g0, kg), pl.ds(base + sub * _SUB, _SUB)], idx_v)

        def conv(j, carry):
            for k in range(kg):
                v = idx_v[k, pl.ds(j * 16, 16)]
                idx_v[k, pl.ds(j * 16, 16)] = v * kmul + k
            return carry

        lax.fori_loop(0, _SUB // 16, conv, 0)
        cps = [
            pltpu.async_copy(tbl_hbm.at[idx_v.at[k]], stg_v.at[k], sem)
            for k in range(kg)
        ]
        for cp in cps:
            cp.wait()

        def row(r, carry):
            acc_lo = jnp.zeros((16,), jnp.float32)
            acc_hi = jnp.zeros((16,), jnp.float32)
            for k in range(kg):
                w = stg_v[k, r, :]
                acc_lo = acc_lo + plsc.bitcast(
                    lax.shift_left(w, shift), jnp.float32)
                acc_hi = acc_hi + plsc.bitcast(
                    lax.bitwise_and(w, mask), jnp.float32)
            acc_v[r, pl.ds(0, 16)] = acc_lo
            acc_v[r, pl.ds(16, 16)] = acc_hi
            return carry

        lax.fori_loop(0, _SUB, row, 0)
        pltpu.sync_copy(acc_v, out_hbm.at[pl.ds(base + sub * _SUB, _SUB)])


def _ln_body(h0_ref, h1_ref, h2_ref, h3_ref, wlin_ref, bconv_ref, blin_ref,
             g_ref, b_ref, out_ref):
    bias = lax.dot_general(bconv_ref[...], wlin_ref[...],
                           (((1,), (1,)), ((), ())),
                           preferred_element_type=jnp.float32) + blin_ref[...]
    x = (h0_ref[...] + h1_ref[...]) + (h2_ref[...] + h3_ref[...]) + bias
    mu = jnp.mean(x, axis=-1, keepdims=True)
    xc = x - mu
    var = jnp.mean(xc * xc, axis=-1, keepdims=True)
    out_ref[...] = xc * lax.rsqrt(var + 1e-5) * g_ref[...] + b_ref[...]


def _make_table(feats, wconv_g, wlin, kg):
    return pl.pallas_call(
        functools.partial(_mat_body, kg),
        grid=(_NBLK,),
        in_specs=[
            pl.BlockSpec((_BLKN, _C), lambda i: (i, 0)),
            pl.BlockSpec((kg, _C, _C), lambda i: (0, 0, 0)),
            pl.BlockSpec((_C, _C), lambda i: (0, 0)),
        ],
        out_specs=pl.BlockSpec((_BLKN, kg * _H), lambda i: (i, 0)),
        out_shape=jax.ShapeDtypeStruct((_N, kg * _H), jnp.int32),
        scratch_shapes=[pltpu.VMEM((_C, kg * _H), jnp.float32),
                        pltpu.VMEM((_C, kg * _H), jnp.float32)],
    )(feats, wconv_g, wlin)


def _gather_group(tbl, idx, kg, g0):
    return pl.kernel(
        functools.partial(_sc_body, kg, g0),
        out_type=jax.ShapeDtypeStruct((_N, _C), jnp.float32),
        mesh=plsc.VectorSubcoreMesh(core_axis_name="c", subcore_axis_name="s"),
        compiler_params=pltpu.CompilerParams(use_tc_tiling_on_sc=False,
                                             needs_layout_passes=False),
        scratch_types=[
            pltpu.VMEM((kg, _SUB), jnp.int32),
            pltpu.VMEM((kg, _SUB, _H), jnp.int32),
            pltpu.VMEM((_SUB, _C), jnp.float32),
            pltpu.SemaphoreType.DMA,
        ],
    )(tbl.reshape(_N * kg, _H), idx)


_GROUPS = (8, 8, 8, 3)


def kernel(feats, neighbor_idx, W_conv, b_conv, W_lin, b_lin, ln_g, ln_b):
    idx = neighbor_idx.astype(jnp.int32)

    hs = []
    g0 = 0
    for kg in _GROUPS:
        tbl = _make_table(feats, W_conv[g0:g0 + kg], W_lin, kg)
        hs.append(_gather_group(tbl, idx, kg, g0))
        g0 += kg

    out = pl.pallas_call(
        _ln_body,
        grid=(_NLNB,),
        in_specs=[
            pl.BlockSpec((_LNB, _C), lambda i: (i, 0)),
            pl.BlockSpec((_LNB, _C), lambda i: (i, 0)),
            pl.BlockSpec((_LNB, _C), lambda i: (i, 0)),
            pl.BlockSpec((_LNB, _C), lambda i: (i, 0)),
            pl.BlockSpec((_C, _C), lambda i: (0, 0)),
            pl.BlockSpec((1, _C), lambda i: (0, 0)),
            pl.BlockSpec((1, _C), lambda i: (0, 0)),
            pl.BlockSpec((1, _C), lambda i: (0, 0)),
            pl.BlockSpec((1, _C), lambda i: (0, 0)),
        ],
        out_specs=pl.BlockSpec((_LNB, _C), lambda i: (i, 0)),
        out_shape=jax.ShapeDtypeStruct((_N, _C), jnp.float32),
    )(*hs, W_lin, b_conv.reshape(1, _C), b_lin.reshape(1, _C),
      ln_g.reshape(1, _C), ln_b.reshape(1, _C))

    return out
